# baseline (speedup 1.0000x reference)
import jax
import jax.numpy as jnp
from jax import lax
from jax.experimental import pallas as pl
from jax.experimental.pallas import tpu as pltpu
from jax.experimental.pallas import tpu_sc as plsc

B, L, K, C = 4, 10000, 16, 256
LANES = 16
CW = C // 2
NW = 32
NROW = B * L
ZROW = NROW
NPAD = 8

PBLK = 80
PNBLK = NROW // PBLK
PB_BASE = PNBLK // NW
PB_EXTRA = PNBLK - PB_BASE * NW
PMAX = PB_BASE + 1

CHUNK = 16
NCHUNK = L // CHUNK
BASE_CNT = NCHUNK // NW
EXTRA = NCHUNK - BASE_CNT * NW
MAX_CNT = BASE_CNT + 1
NITEM = MAX_CNT * B
NPAIR = NITEM // 2
GROUP = CHUNK * K

_PACKFMT = plsc.PackFormat.INTERLEAVED


def _max_tree(vals):
    while len(vals) > 1:
        vals = [jnp.maximum(a, b) for a, b in zip(vals[::2], vals[1::2])] + (
            [vals[-1]] if len(vals) % 2 else [])
    return vals[0]


def _wid():
    return lax.axis_index("subcore") * 2 + lax.axis_index("core")


def _pack_body(src_hbm, ptab_hbm, src_a, src_b, pstage_a, pstage_b,
               sem_a, sem_b, osem_a, osem_b):
    wid = _wid()
    start_blk = wid * PB_BASE + jnp.minimum(wid, PB_EXTRA)
    cnt = PB_BASE + jnp.where(wid < PB_EXTRA, 1, 0)

    def _off(t):
        blk = jnp.minimum(start_blk + t, PNBLK - 1)
        return pl.multiple_of(blk * PBLK, PBLK)

    def start(t, buf, sem):
        pltpu.async_copy(src_hbm.at[pl.ds(_off(t), PBLK), :], buf, sem)

    def wait(t, buf, sem):
        pltpu.make_async_copy(src_hbm.at[pl.ds(_off(t), PBLK), :],
                              buf, sem).wait()

    def store_wait(t, ps, osem):
        @pl.when((t >= 0) & (t < cnt))
        def _():
            pltpu.make_async_copy(ps, ptab_hbm.at[pl.ds(_off(t), PBLK), :],
                                  osem).wait()

    def pack_block(t, buf, ps, osem):
        store_wait(t - 2, ps, osem)

        @pl.loop(0, PBLK)
        def _(r):
            for cc in range(C // 32):
                a = buf[r, pl.ds(cc * 32, LANES)]
                b = buf[r, pl.ds(cc * 32 + LANES, LANES)]
                w = plsc.pack(a, b, format=_PACKFMT)
                ps[r, pl.ds(cc * LANES, LANES)] = plsc.bitcast(w, jnp.int32)

        @pl.when(t < cnt)
        def _():
            pltpu.async_copy(ps, ptab_hbm.at[pl.ds(_off(t), PBLK), :], osem)

    @pl.when(wid == 0)
    def _():
        zero = jnp.zeros((LANES,), jnp.int32)
        for r in range(NPAD):
            for cc in range(CW // LANES):
                pstage_a[r, pl.ds(cc * LANES, LANES)] = zero
        pltpu.sync_copy(pstage_a.at[pl.ds(0, NPAD), :],
                        ptab_hbm.at[pl.ds(NROW, NPAD), :])

    start(0, src_a, sem_a)

    @pl.loop(0, PMAX // 2)
    def _(j):
        t0 = j * 2
        start(t0 + 1, src_b, sem_b)
        wait(t0, src_a, sem_a)
        pack_block(t0, src_a, pstage_a, osem_a)

        @pl.when(j < PMAX // 2 - 1)
        def _():
            start(t0 + 2, src_a, sem_a)

        wait(t0 + 1, src_b, sem_b)
        pack_block(t0 + 1, src_b, pstage_b, osem_b)

    store_wait(jnp.int32(PMAX - 2), pstage_a, osem_a)
    store_wait(jnp.int32(PMAX - 1), pstage_b, osem_b)


HGROUP = GROUP // 2


def _gather_body(table_hbm, idx_hbm, out_hbm,
                 idx_raw, idxb_a1, idxb_a2, idxb_b1, idxb_b2,
                 rows_a1, rows_a2, rows_b1, rows_b2, ostage,
                 sem_a1, sem_a2, sem_b1, sem_b2):
    wid = _wid()
    start = wid * BASE_CNT + jnp.minimum(wid, EXTRA)
    count = BASE_CNT + jnp.where(wid < EXTRA, 1, 0)

    def prepare(t, x1, x2, r1, r2, s1, s2):
        b = lax.rem(t, B)
        chunk = jnp.minimum(start + lax.div(t, B), NCHUNK - 1)

        @pl.when(b == 0)
        def _():
            pltpu.sync_copy(idx_hbm.at[chunk], idx_raw)

        boff = b * L
        for half, idxb_x in ((0, x1), (1, x2)):
            for j in range(HGROUP // LANES):
                v = idx_raw[pl.ds(half * HGROUP + j * LANES, LANES)]
                vb = jnp.where(v >= 0, v + boff,
                               jnp.full((LANES,), ZROW, jnp.int32))
                idxb_x[pl.ds(j * LANES, LANES)] = vb
        pltpu.async_copy(table_hbm.at[x1], r1, s1)
        pltpu.async_copy(table_hbm.at[x2], r2, s2)

    def wait(x1, x2, r1, r2, s1, s2):
        pltpu.make_async_copy(table_hbm.at[x1], r1, s1).wait()
        pltpu.make_async_copy(table_hbm.at[x2], r2, s2).wait()

    def compute(t, r1, r2):
        b = lax.rem(t, B)
        local = lax.div(t, B)
        chunk = jnp.minimum(start + local, NCHUNK - 1)
        row0 = pl.multiple_of(b * L + chunk * CHUNK, CHUNK)

        for half, rows_x in ((0, r1), (1, r2)):
            @pl.loop(0, CHUNK // 2)
            def _(r):
                base = r * K
                for cc in range(CW // LANES):
                    sl = pl.ds(cc * LANES, LANES)
                    vals = [plsc.bitcast(rows_x[base + k, sl], jnp.bfloat16)
                            for k in range(K)]
                    hi, lo = plsc.unpack(_max_tree(vals), format=_PACKFMT)
                    ostage[half * (CHUNK // 2) + r, pl.ds(cc * 32, LANES)] = hi
                    ostage[half * (CHUNK // 2) + r,
                           pl.ds(cc * 32 + LANES, LANES)] = lo

        @pl.when(local < count)
        def _():
            pltpu.sync_copy(ostage, out_hbm.at[pl.ds(row0, CHUNK), :])

    prepare(jnp.int32(0), idxb_a1, idxb_a2, rows_a1, rows_a2, sem_a1, sem_a2)

    @pl.loop(0, NPAIR)
    def _(j):
        t0 = j * 2
        wait(idxb_a1, idxb_a2, rows_a1, rows_a2, sem_a1, sem_a2)
        prepare(t0 + 1, idxb_b1, idxb_b2, rows_b1, rows_b2, sem_b1, sem_b2)
        compute(t0, rows_a1, rows_a2)
        wait(idxb_b1, idxb_b2, rows_b1, rows_b2, sem_b1, sem_b2)

        @pl.when(j < NPAIR - 1)
        def _():
            prepare(t0 + 2, idxb_a1, idxb_a2, rows_a1, rows_a2,
                    sem_a1, sem_a2)

        compute(t0 + 1, rows_b1, rows_b2)


def kernel(inputs, neighbor_indices):
    src = inputs.reshape(NROW, C)
    idx2 = neighbor_indices.reshape(NCHUNK, GROUP)
    mesh = plsc.VectorSubcoreMesh(core_axis_name="core",
                                  subcore_axis_name="subcore")
    cp = pltpu.CompilerParams(needs_layout_passes=False)

    pack_k = pl.kernel(
        _pack_body,
        out_type=jax.ShapeDtypeStruct((NROW + NPAD, CW), jnp.int32),
        mesh=mesh,
        compiler_params=cp,
        scratch_types=[
            pltpu.VMEM((PBLK, C), jnp.float32),
            pltpu.VMEM((PBLK, C), jnp.float32),
            pltpu.VMEM((PBLK, CW), jnp.int32),
            pltpu.VMEM((PBLK, CW), jnp.int32),
            pltpu.SemaphoreType.DMA,
            pltpu.SemaphoreType.DMA,
            pltpu.SemaphoreType.DMA,
            pltpu.SemaphoreType.DMA,
        ],
    )
    table = pack_k(src)

    gather_k = pl.kernel(
        _gather_body,
        out_type=jax.ShapeDtypeStruct((NROW, C), jnp.float32),
        mesh=mesh,
        compiler_params=cp,
        scratch_types=[
            pltpu.VMEM((GROUP,), jnp.int32),
            pltpu.VMEM((HGROUP,), jnp.int32),
            pltpu.VMEM((HGROUP,), jnp.int32),
            pltpu.VMEM((HGROUP,), jnp.int32),
            pltpu.VMEM((HGROUP,), jnp.int32),
            pltpu.VMEM((HGROUP, CW), jnp.int32),
            pltpu.VMEM((HGROUP, CW), jnp.int32),
            pltpu.VMEM((HGROUP, CW), jnp.int32),
            pltpu.VMEM((HGROUP, CW), jnp.int32),
            pltpu.VMEM((CHUNK, C), jnp.float32),
            pltpu.SemaphoreType.DMA,
            pltpu.SemaphoreType.DMA,
            pltpu.SemaphoreType.DMA,
            pltpu.SemaphoreType.DMA,
        ],
    )
    out = gather_k(table, idx2)
    return out.reshape(B, L, C)


# speedup vs baseline: 1.0229x; 1.0229x over previous
import jax
import jax.numpy as jnp
from jax import lax
from jax.experimental import pallas as pl
from jax.experimental.pallas import tpu as pltpu
from jax.experimental.pallas import tpu_sc as plsc

B, L, K, C = 4, 10000, 16, 256
LANES = 16
CW = C // 2
NW = 32
NROW = B * L
ZROW = NROW
NPAD = 8

PBLK = 160
PNBLK = NROW // PBLK
PB_BASE = PNBLK // NW
PB_EXTRA = PNBLK - PB_BASE * NW
PMAX = PB_BASE + 1

CHUNK = 16
NCHUNK = L // CHUNK
BASE_CNT = NCHUNK // NW
EXTRA = NCHUNK - BASE_CNT * NW
MAX_CNT = BASE_CNT + 1
NITEM = MAX_CNT * B
NPAIR = NITEM // 2
GROUP = CHUNK * K

_PACKFMT = plsc.PackFormat.INTERLEAVED


def _max_tree(vals):
    while len(vals) > 1:
        vals = [jnp.maximum(a, b) for a, b in zip(vals[::2], vals[1::2])] + (
            [vals[-1]] if len(vals) % 2 else [])
    return vals[0]


def _wid():
    return lax.axis_index("subcore") * 2 + lax.axis_index("core")


def _pack_body(src_hbm, ptab_hbm, src_a, src_b, pstage_a, pstage_b,
               sem_a, sem_b, osem_a, osem_b):
    wid = _wid()
    start_blk = wid * PB_BASE + jnp.minimum(wid, PB_EXTRA)
    cnt = PB_BASE + jnp.where(wid < PB_EXTRA, 1, 0)

    def _off(t):
        blk = jnp.minimum(start_blk + t, PNBLK - 1)
        return pl.multiple_of(blk * PBLK, PBLK)

    def start(t, buf, sem):
        pltpu.async_copy(src_hbm.at[pl.ds(_off(t), PBLK), :], buf, sem)

    def wait(t, buf, sem):
        pltpu.make_async_copy(src_hbm.at[pl.ds(_off(t), PBLK), :],
                              buf, sem).wait()

    def store_wait(t, ps, osem):
        @pl.when((t >= 0) & (t < cnt))
        def _():
            pltpu.make_async_copy(ps, ptab_hbm.at[pl.ds(_off(t), PBLK), :],
                                  osem).wait()

    def pack_block(t, buf, ps, osem):
        store_wait(t - 2, ps, osem)

        @pl.loop(0, PBLK)
        def _(r):
            for cc in range(C // 32):
                a = buf[r, pl.ds(cc * 32, LANES)]
                b = buf[r, pl.ds(cc * 32 + LANES, LANES)]
                w = plsc.pack(a, b, format=_PACKFMT)
                ps[r, pl.ds(cc * LANES, LANES)] = plsc.bitcast(w, jnp.int32)

        @pl.when(t < cnt)
        def _():
            pltpu.async_copy(ps, ptab_hbm.at[pl.ds(_off(t), PBLK), :], osem)

    @pl.when(wid == 0)
    def _():
        zero = jnp.zeros((LANES,), jnp.int32)
        for r in range(NPAD):
            for cc in range(CW // LANES):
                pstage_a[r, pl.ds(cc * LANES, LANES)] = zero
        pltpu.sync_copy(pstage_a.at[pl.ds(0, NPAD), :],
                        ptab_hbm.at[pl.ds(NROW, NPAD), :])

    start(0, src_a, sem_a)

    @pl.loop(0, PMAX // 2)
    def _(j):
        t0 = j * 2
        start(t0 + 1, src_b, sem_b)
        wait(t0, src_a, sem_a)
        pack_block(t0, src_a, pstage_a, osem_a)

        @pl.when(j < PMAX // 2 - 1)
        def _():
            start(t0 + 2, src_a, sem_a)

        wait(t0 + 1, src_b, sem_b)
        pack_block(t0 + 1, src_b, pstage_b, osem_b)

    store_wait(jnp.int32(PMAX - 2), pstage_a, osem_a)
    store_wait(jnp.int32(PMAX - 1), pstage_b, osem_b)


def _gather_body(table_hbm, idx_hbm, out_hbm,
                 idx_raw, idxb_a, idxb_b, rows_a, rows_b, ostage_a, ostage_b,
                 sem_a, sem_b, osem_a, osem_b):
    wid = _wid()
    start = wid * BASE_CNT + jnp.minimum(wid, EXTRA)
    count = BASE_CNT + jnp.where(wid < EXTRA, 1, 0)

    def _orow(t):
        b = lax.rem(t, B)
        local = lax.div(t, B)
        chunk = jnp.minimum(start + local, NCHUNK - 1)
        return pl.multiple_of(b * L + chunk * CHUNK, CHUNK)

    def prepare(t, idxb_x, rows_x, sem_x):
        b = lax.rem(t, B)
        chunk = jnp.minimum(start + lax.div(t, B), NCHUNK - 1)

        @pl.when(b == 0)
        def _():
            pltpu.sync_copy(idx_hbm.at[chunk], idx_raw)

        boff = b * L
        for j in range(GROUP // LANES):
            v = idx_raw[pl.ds(j * LANES, LANES)]
            vb = jnp.where(v >= 0, v + boff, jnp.full((LANES,), ZROW, jnp.int32))
            idxb_x[pl.ds(j * LANES, LANES)] = vb
        pltpu.async_copy(table_hbm.at[idxb_x], rows_x, sem_x)

    def wait(idxb_x, rows_x, sem_x):
        pltpu.make_async_copy(table_hbm.at[idxb_x], rows_x, sem_x).wait()

    def out_wait(t, os, osem):
        @pl.when((t >= 0) & (lax.div(t, B) < count))
        def _():
            pltpu.make_async_copy(os, out_hbm.at[pl.ds(_orow(t), CHUNK), :],
                                  osem).wait()

    def compute(t, rows_x, os, osem):
        local = lax.div(t, B)
        out_wait(t - 2, os, osem)

        @pl.loop(0, CHUNK)
        def _(r):
            base = r * K
            for cc in range(CW // LANES):
                sl = pl.ds(cc * LANES, LANES)
                vals = [plsc.bitcast(rows_x[base + k, sl], jnp.bfloat16)
                        for k in range(K)]
                hi, lo = plsc.unpack(_max_tree(vals), format=_PACKFMT)
                os[r, pl.ds(cc * 32, LANES)] = hi
                os[r, pl.ds(cc * 32 + LANES, LANES)] = lo

        @pl.when(local < count)
        def _():
            pltpu.async_copy(os, out_hbm.at[pl.ds(_orow(t), CHUNK), :], osem)

    prepare(jnp.int32(0), idxb_a, rows_a, sem_a)

    @pl.loop(0, NPAIR)
    def _(j):
        t0 = j * 2
        wait(idxb_a, rows_a, sem_a)
        prepare(t0 + 1, idxb_b, rows_b, sem_b)
        compute(t0, rows_a, ostage_a, osem_a)
        wait(idxb_b, rows_b, sem_b)

        @pl.when(j < NPAIR - 1)
        def _():
            prepare(t0 + 2, idxb_a, rows_a, sem_a)

        compute(t0 + 1, rows_b, ostage_b, osem_b)

    out_wait(jnp.int32(NITEM - 2), ostage_a, osem_a)
    out_wait(jnp.int32(NITEM - 1), ostage_b, osem_b)


def kernel(inputs, neighbor_indices):
    src = inputs.reshape(NROW, C)
    idx2 = neighbor_indices.reshape(NCHUNK, GROUP)
    mesh = plsc.VectorSubcoreMesh(core_axis_name="core",
                                  subcore_axis_name="subcore")
    cp = pltpu.CompilerParams(needs_layout_passes=False)

    pack_k = pl.kernel(
        _pack_body,
        out_type=jax.ShapeDtypeStruct((NROW + NPAD, CW), jnp.int32),
        mesh=mesh,
        compiler_params=cp,
        scratch_types=[
            pltpu.VMEM((PBLK, C), jnp.float32),
            pltpu.VMEM((PBLK, C), jnp.float32),
            pltpu.VMEM((PBLK, CW), jnp.int32),
            pltpu.VMEM((PBLK, CW), jnp.int32),
            pltpu.SemaphoreType.DMA,
            pltpu.SemaphoreType.DMA,
            pltpu.SemaphoreType.DMA,
            pltpu.SemaphoreType.DMA,
        ],
    )
    table = pack_k(src)

    gather_k = pl.kernel(
        _gather_body,
        out_type=jax.ShapeDtypeStruct((NROW, C), jnp.float32),
        mesh=mesh,
        compiler_params=cp,
        scratch_types=[
            pltpu.VMEM((GROUP,), jnp.int32),
            pltpu.VMEM((GROUP,), jnp.int32),
            pltpu.VMEM((GROUP,), jnp.int32),
            pltpu.VMEM((GROUP, CW), jnp.int32),
            pltpu.VMEM((GROUP, CW), jnp.int32),
            pltpu.VMEM((CHUNK, C), jnp.float32),
            pltpu.VMEM((CHUNK, C), jnp.float32),
            pltpu.SemaphoreType.DMA,
            pltpu.SemaphoreType.DMA,
            pltpu.SemaphoreType.DMA,
            pltpu.SemaphoreType.DMA,
        ],
    )
    out = gather_k(table, idx2)
    return out.reshape(B, L, C)


# speedup vs baseline: 1.0950x; 1.0705x over previous
import jax
import jax.numpy as jnp
from jax import lax
from jax.experimental import pallas as pl
from jax.experimental.pallas import tpu as pltpu
from jax.experimental.pallas import tpu_sc as plsc

B, L, K, C = 4, 10000, 16, 256
LANES = 16
CW = C // 2
NW = 32
NROW = B * L
ZROW = NROW
NPAD = 8

PBLK = 160
PNBLK = NROW // PBLK
PB_BASE = PNBLK // NW
PB_EXTRA = PNBLK - PB_BASE * NW
PMAX = PB_BASE + 1

CHUNK = 16
NCHUNK = L // CHUNK
BASE_CNT = NCHUNK // NW
EXTRA = NCHUNK - BASE_CNT * NW
MAX_CNT = BASE_CNT + 1
NITEM = MAX_CNT * B
NPAIR = NITEM // 2
GROUP = CHUNK * K

_PACKFMT = plsc.PackFormat.INTERLEAVED


def _max_tree(vals):
    while len(vals) > 1:
        vals = [jnp.maximum(a, b) for a, b in zip(vals[::2], vals[1::2])] + (
            [vals[-1]] if len(vals) % 2 else [])
    return vals[0]


def _wid():
    return lax.axis_index("subcore") * 2 + lax.axis_index("core")


def _pack_body(src_hbm, ptab_hbm, src_a, src_b, pstage_a, pstage_b,
               sem_a, sem_b, osem_a, osem_b):
    wid = _wid()
    start_blk = wid * PB_BASE + jnp.minimum(wid, PB_EXTRA)
    cnt = PB_BASE + jnp.where(wid < PB_EXTRA, 1, 0)

    def _off(t):
        blk = jnp.minimum(start_blk + t, PNBLK - 1)
        return pl.multiple_of(blk * PBLK, PBLK)

    def start(t, buf, sem):
        pltpu.async_copy(src_hbm.at[pl.ds(_off(t), PBLK), :], buf, sem)

    def wait(t, buf, sem):
        pltpu.make_async_copy(src_hbm.at[pl.ds(_off(t), PBLK), :],
                              buf, sem).wait()

    def store_wait(t, ps, osem):
        @pl.when((t >= 0) & (t < cnt))
        def _():
            pltpu.make_async_copy(ps, ptab_hbm.at[pl.ds(_off(t), PBLK), :],
                                  osem).wait()

    def pack_block(t, buf, ps, osem):
        store_wait(t - 2, ps, osem)

        @pl.loop(0, PBLK)
        def _(r):
            for cc in range(C // 32):
                a = buf[r, pl.ds(cc * 32, LANES)]
                b = buf[r, pl.ds(cc * 32 + LANES, LANES)]
                w = plsc.pack(a, b, format=_PACKFMT)
                ps[r, pl.ds(cc * LANES, LANES)] = plsc.bitcast(w, jnp.int32)

        @pl.when(t < cnt)
        def _():
            pltpu.async_copy(ps, ptab_hbm.at[pl.ds(_off(t), PBLK), :], osem)

    @pl.when(wid == 0)
    def _():
        zero = jnp.zeros((LANES,), jnp.int32)
        for r in range(NPAD):
            for cc in range(CW // LANES):
                pstage_a[r, pl.ds(cc * LANES, LANES)] = zero
        pltpu.sync_copy(pstage_a.at[pl.ds(0, NPAD), :],
                        ptab_hbm.at[pl.ds(NROW, NPAD), :])

    start(0, src_a, sem_a)

    @pl.loop(0, PMAX // 2)
    def _(j):
        t0 = j * 2
        start(t0 + 1, src_b, sem_b)
        wait(t0, src_a, sem_a)
        pack_block(t0, src_a, pstage_a, osem_a)

        @pl.when(j < PMAX // 2 - 1)
        def _():
            start(t0 + 2, src_a, sem_a)

        wait(t0 + 1, src_b, sem_b)
        pack_block(t0 + 1, src_b, pstage_b, osem_b)

    store_wait(jnp.int32(PMAX - 2), pstage_a, osem_a)
    store_wait(jnp.int32(PMAX - 1), pstage_b, osem_b)


def _gather_body(table_hbm, idx_hbm, out_hbm,
                 idx_raw, idxb_a, idxb_b, rows_a, rows_b, ostage_a, ostage_b,
                 sem_a, sem_b, osem_a, osem_b):
    wid = _wid()
    start = wid * BASE_CNT + jnp.minimum(wid, EXTRA)
    count = BASE_CNT + jnp.where(wid < EXTRA, 1, 0)

    def _orow(t):
        b = lax.rem(t, B)
        local = lax.div(t, B)
        chunk = jnp.minimum(start + local, NCHUNK - 1)
        return pl.multiple_of(b * L + chunk * CHUNK, CHUNK)

    def prepare(t, idxb_x, rows_x, sem_x):
        b = lax.rem(t, B)
        chunk = jnp.minimum(start + lax.div(t, B), NCHUNK - 1)

        @pl.when(b == 0)
        def _():
            pltpu.sync_copy(idx_hbm.at[chunk], idx_raw)

        boff = b * L
        for j in range(GROUP // LANES):
            v = idx_raw[pl.ds(j * LANES, LANES)]
            vb = jnp.where(v >= 0, v + boff, jnp.full((LANES,), ZROW, jnp.int32))
            idxb_x[pl.ds(j * LANES, LANES)] = vb
        pltpu.async_copy(table_hbm.at[idxb_x], rows_x, sem_x)

    def wait(idxb_x, rows_x, sem_x):
        pltpu.make_async_copy(table_hbm.at[idxb_x], rows_x, sem_x).wait()

    def out_wait(t, os, osem):
        @pl.when((t >= 0) & (lax.div(t, B) < count))
        def _():
            pltpu.make_async_copy(os, out_hbm.at[pl.ds(_orow(t), CHUNK), :],
                                  osem).wait()

    def compute(t, rows_x, os, osem):
        local = lax.div(t, B)
        out_wait(t - 2, os, osem)

        @pl.loop(0, CHUNK)
        def _(r):
            base = r * K
            for cc in range(CW // LANES):
                sl = pl.ds(cc * LANES, LANES)
                vals = [plsc.bitcast(rows_x[base + k, sl], jnp.bfloat16)
                        for k in range(K)]
                lo, hi = plsc.unpack(_max_tree(vals), format=_PACKFMT)
                os[r, pl.ds(cc * LANES, LANES)] = lo
                os[r, pl.ds(cc * LANES + CW, LANES)] = hi

        @pl.when(local < count)
        def _():
            pltpu.async_copy(os, out_hbm.at[pl.ds(_orow(t), CHUNK), :], osem)

    prepare(jnp.int32(0), idxb_a, rows_a, sem_a)

    @pl.loop(0, NPAIR)
    def _(j):
        t0 = j * 2
        wait(idxb_a, rows_a, sem_a)
        prepare(t0 + 1, idxb_b, rows_b, sem_b)
        compute(t0, rows_a, ostage_a, osem_a)
        wait(idxb_b, rows_b, sem_b)

        @pl.when(j < NPAIR - 1)
        def _():
            prepare(t0 + 2, idxb_a, rows_a, sem_a)

        compute(t0 + 1, rows_b, ostage_b, osem_b)

    out_wait(jnp.int32(NITEM - 2), ostage_a, osem_a)
    out_wait(jnp.int32(NITEM - 1), ostage_b, osem_b)


def kernel(inputs, neighbor_indices):
    idx2 = neighbor_indices.reshape(NCHUNK, GROUP)
    mesh = plsc.VectorSubcoreMesh(core_axis_name="core",
                                  subcore_axis_name="subcore")
    cp = pltpu.CompilerParams(needs_layout_passes=False)

    xb = inputs.astype(jnp.bfloat16).reshape(NROW, C)
    u = jax.lax.bitcast_convert_type(xb, jnp.uint16)
    w = (u[:, :CW].astype(jnp.int32)
         | (u[:, CW:].astype(jnp.int32) << 16))
    table = jnp.concatenate([w, jnp.zeros((NPAD, CW), jnp.int32)], axis=0)

    gather_k = pl.kernel(
        _gather_body,
        out_type=jax.ShapeDtypeStruct((NROW, C), jnp.float32),
        mesh=mesh,
        compiler_params=cp,
        scratch_types=[
            pltpu.VMEM((GROUP,), jnp.int32),
            pltpu.VMEM((GROUP,), jnp.int32),
            pltpu.VMEM((GROUP,), jnp.int32),
            pltpu.VMEM((GROUP, CW), jnp.int32),
            pltpu.VMEM((GROUP, CW), jnp.int32),
            pltpu.VMEM((CHUNK, C), jnp.float32),
            pltpu.VMEM((CHUNK, C), jnp.float32),
            pltpu.SemaphoreType.DMA,
            pltpu.SemaphoreType.DMA,
            pltpu.SemaphoreType.DMA,
            pltpu.SemaphoreType.DMA,
        ],
    )
    out = gather_k(table, idx2)
    return out.reshape(B, L, C)
